# Initial kernel scaffold; baseline (speedup 1.0000x reference)
import jax
import jax.numpy as jnp
from jax.experimental import pallas as pl


def kernel(boxes, scores, labels, img_size, nms_thresh, class_weights):
    raise NotImplementedError("write your pallas kernel here")


# speedup vs baseline: 1013.5817x; 1013.5817x over previous
import functools

import jax
import jax.numpy as jnp
from jax import lax
from jax.experimental import pallas as pl
from jax.experimental.pallas import tpu as pltpu

_MAX_DET = 300
_SCORE_THRESH = 0.3


def _nms_body(B, Np, scal_ref,
              x1_ref, y1_ref, x2_ref, y2_ref, sc_ref, lb_ref,
              ob1_o, ob2_o, ob3_o, ob4_o, osc_o, olb_o,
              ox1_s, oy1_s, ox2_s, oy2_s, aj_s, alive_s):
    imgf = scal_ref[0, 0]
    thr = scal_ref[0, 1]
    cw0 = scal_ref[0, 2]
    cw1 = scal_ref[0, 3]

    lb = lb_ref[...]
    lbf = lb.astype(jnp.float32)
    sc = sc_ref[...]
    weighted = sc * jnp.where(lb == 0, cw0, cw1)
    valid = sc > _SCORE_THRESH
    off = lbf * (imgf + 1.0)
    ox1 = jnp.clip(x1_ref[...], 0.0, imgf) + off
    oy1 = jnp.clip(y1_ref[...], 0.0, imgf) + off
    ox2 = jnp.clip(x2_ref[...], 0.0, imgf) + off
    oy2 = jnp.clip(y2_ref[...], 0.0, imgf) + off
    ox1_s[...] = ox1
    oy1_s[...] = oy1
    ox2_s[...] = ox2
    oy2_s[...] = oy2
    aj_s[...] = (ox2 - ox1) * (oy2 - oy1)
    alive_s[...] = jnp.where(valid, weighted, -jnp.inf)

    ob1_o[...] = jnp.zeros_like(ob1_o)
    ob2_o[...] = jnp.zeros_like(ob2_o)
    ob3_o[...] = jnp.zeros_like(ob3_o)
    ob4_o[...] = jnp.zeros_like(ob4_o)
    osc_o[...] = jnp.zeros_like(osc_o)
    olb_o[...] = jnp.full_like(olb_o, -1)

    idx = lax.broadcasted_iota(jnp.int32, (B, Np), 1)
    pos = lax.broadcasted_iota(jnp.int32, (B, _MAX_DET), 1)

    def step(t, counts):
        alive = alive_s[...]
        m = jnp.max(alive, axis=1, keepdims=True)
        active = m > -jnp.inf
        ismax = alive == m
        win = jnp.min(jnp.where(ismax, idx, Np), axis=1, keepdims=True)
        onehot = idx == win

        ox1v = ox1_s[...]
        oy1v = oy1_s[...]
        ox2v = ox2_s[...]
        oy2v = oy2_s[...]

        def extf(a):
            return jnp.sum(jnp.where(onehot, a, 0.0), axis=1, keepdims=True)

        wx1 = extf(ox1v)
        wy1 = extf(oy1v)
        wx2 = extf(ox2v)
        wy2 = extf(oy2v)
        wsc = extf(sc_ref[...])
        wlb = jnp.sum(jnp.where(onehot, lb_ref[...], 0), axis=1, keepdims=True)

        xx1 = jnp.maximum(wx1, ox1v)
        yy1 = jnp.maximum(wy1, oy1v)
        xx2 = jnp.minimum(wx2, ox2v)
        yy2 = jnp.minimum(wy2, oy2v)
        inter = jnp.maximum(xx2 - xx1, 0.0) * jnp.maximum(yy2 - yy1, 0.0)
        a_i = (wx2 - wx1) * (wy2 - wy1)
        iou = inter / (a_i + aj_s[...] - inter + 1e-9)
        kill = (iou > thr) | onehot
        alive_s[...] = jnp.where(kill & active, -jnp.inf, alive)

        posoh = (pos == counts) & active
        woff = wlb.astype(jnp.float32) * (imgf + 1.0)
        ob1_o[...] = jnp.where(posoh, wx1 - woff, ob1_o[...])
        ob2_o[...] = jnp.where(posoh, wy1 - woff, ob2_o[...])
        ob3_o[...] = jnp.where(posoh, wx2 - woff, ob3_o[...])
        ob4_o[...] = jnp.where(posoh, wy2 - woff, ob4_o[...])
        osc_o[...] = jnp.where(posoh, wsc, osc_o[...])
        olb_o[...] = jnp.where(posoh, wlb, olb_o[...])
        return counts + jnp.where(active, 1, 0)

    lax.fori_loop(0, _MAX_DET, step, jnp.zeros((B, 1), jnp.int32))


def kernel(boxes, scores, labels, img_size, nms_thresh, class_weights):
    B, N, _ = boxes.shape
    Np = ((N + 127) // 128) * 128
    pad = Np - N

    thr = jax.nn.sigmoid(jnp.asarray(nms_thresh, jnp.float32))
    imgf = jnp.asarray(img_size, jnp.float32)
    cw = jnp.asarray(class_weights, jnp.float32)
    scal = jnp.stack([imgf, thr, cw[0], cw[1]]).reshape(1, 4)

    x1 = jnp.pad(boxes[:, :, 0], ((0, 0), (0, pad)))
    y1 = jnp.pad(boxes[:, :, 1], ((0, 0), (0, pad)))
    x2 = jnp.pad(boxes[:, :, 2], ((0, 0), (0, pad)))
    y2 = jnp.pad(boxes[:, :, 3], ((0, 0), (0, pad)))
    scp = jnp.pad(scores, ((0, 0), (0, pad)), constant_values=-1.0)
    lbp = jnp.pad(labels.astype(jnp.int32), ((0, 0), (0, pad)))

    fshape = jax.ShapeDtypeStruct((B, _MAX_DET), jnp.float32)
    ishape = jax.ShapeDtypeStruct((B, _MAX_DET), jnp.int32)
    bx1, by1, bx2, by2, osc, olb = pl.pallas_call(
        functools.partial(_nms_body, B, Np),
        in_specs=[
            pl.BlockSpec(memory_space=pltpu.SMEM),
            pl.BlockSpec(memory_space=pltpu.VMEM),
            pl.BlockSpec(memory_space=pltpu.VMEM),
            pl.BlockSpec(memory_space=pltpu.VMEM),
            pl.BlockSpec(memory_space=pltpu.VMEM),
            pl.BlockSpec(memory_space=pltpu.VMEM),
            pl.BlockSpec(memory_space=pltpu.VMEM),
        ],
        out_shape=[fshape, fshape, fshape, fshape, fshape, ishape],
        scratch_shapes=[pltpu.VMEM((B, Np), jnp.float32)] * 6,
    )(scal, x1, y1, x2, y2, scp, lbp)

    out_boxes = jnp.stack([bx1, by1, bx2, by2], axis=-1)
    return out_boxes, osc, olb
